# Initial kernel scaffold; baseline (speedup 1.0000x reference)
import jax
import jax.numpy as jnp
from jax.experimental import pallas as pl


def kernel(params, cell_idx, edge_index_intra_diff, edge_index_inter_diff, edge_index_intra_lt, edge_index_intra_gt):
    raise NotImplementedError("write your pallas kernel here")


# speedup vs baseline: 1.5329x; 1.5329x over previous
import functools

import jax
import jax.numpy as jnp
from jax import lax
from jax.experimental import pallas as pl
from jax.experimental.pallas import tpu as pltpu
from jax.experimental.pallas import tpu_sc as plsc

N = 10000
H = 128
E = 40000
NET = 4
STEPS = 2

NB = 1000
EB = 1000
CH = 96
NWORK = 32
NCH = 13
PER_W = NCH * CH
TAIL = E - NWORK * PER_W
E_HALF = E // 2
PER_S = NCH * CH
SC_TAIL = E_HALF - 16 * PER_S
ROWS_PER_S = 624
ROW_TAIL = N - 16 * ROWS_PER_S
ZR = 16

_f32 = jnp.float32


def _embed_body(ci_ref, emb_ref, x_ref):
    idx = ci_ref[...]
    acc = jnp.zeros((NB, H), _f32)
    for k in range(3):
        acc = acc + jnp.where(idx == k, 1.0, 0.0) * emb_ref[k][None, :]
    x_ref[...] = acc


def _embed(ci2, embp):
    return pl.pallas_call(
        _embed_body,
        grid=(N // NB,),
        in_specs=[
            pl.BlockSpec((NB, H), lambda t: (t, 0)),
            pl.BlockSpec((8, H), lambda t: (0, 0)),
        ],
        out_specs=pl.BlockSpec((NB, H), lambda t: (t, 0)),
        out_shape=jax.ShapeDtypeStruct((N, H), _f32),
    )(ci2, embp)


def _pq_body(h_ref, w_ref, *out_refs):
    acc = jnp.dot(h_ref[...], w_ref[...], preferred_element_type=_f32)
    for i in range(2 * NET):
        out_refs[i][...] = acc[:, i * H:(i + 1) * H]


def _pq(h, wpq):
    return pl.pallas_call(
        _pq_body,
        grid=(N // NB,),
        in_specs=[
            pl.BlockSpec((NB, H), lambda t: (t, 0)),
            pl.BlockSpec((H, 2 * NET * H), lambda t: (0, 0)),
        ],
        out_specs=[pl.BlockSpec((NB, H), lambda t: (t, 0))] * (2 * NET),
        out_shape=[jax.ShapeDtypeStruct((N, H), _f32)] * (2 * NET),
    )(h, wpq)


def _gather_body(p0, p1, p2, p3, q0, q1, q2, q3,
                 s0, s1, s2, s3, d0, d1, d2, d3,
                 ps_out, qd_out,
                 isrc, idst, rp, rq, its, itd, rtp, rtq, sem):
    c = lax.axis_index("c")
    s = lax.axis_index("s")
    w = s * 2 + c
    base_w = w * PER_W
    ptabs = (p0, p1, p2, p3)
    qtabs = (q0, q1, q2, q3)
    srcs = (s0, s1, s2, s3)
    dsts = (d0, d1, d2, d3)
    for et in range(NET):
        def chunk(j, carry, et=et):
            base = base_w + j * CH
            pltpu.sync_copy(srcs[et].at[pl.ds(base, CH)], isrc)
            pltpu.sync_copy(dsts[et].at[pl.ds(base, CH)], idst)
            pltpu.async_copy(ptabs[et].at[isrc], rp, sem).wait()
            pltpu.async_copy(qtabs[et].at[idst], rq, sem).wait()
            pltpu.sync_copy(rp, ps_out.at[et, pl.ds(base, CH)])
            pltpu.sync_copy(rq, qd_out.at[et, pl.ds(base, CH)])
            return carry
        lax.fori_loop(0, NCH, chunk, 0)

        @pl.when(w < TAIL // 8)
        def _(et=et):
            tb = NWORK * PER_W + w * 8
            pltpu.sync_copy(srcs[et].at[pl.ds(tb, 8)], its)
            pltpu.sync_copy(dsts[et].at[pl.ds(tb, 8)], itd)
            pltpu.async_copy(ptabs[et].at[its], rtp, sem).wait()
            pltpu.async_copy(qtabs[et].at[itd], rtq, sem).wait()
            pltpu.sync_copy(rtp, ps_out.at[et, pl.ds(tb, 8)])
            pltpu.sync_copy(rtq, qd_out.at[et, pl.ds(tb, 8)])


def _gather(pqs, srcs, dsts):
    f = functools.partial(
        pl.kernel,
        out_type=[jax.ShapeDtypeStruct((NET, E, H), _f32)] * 2,
        mesh=plsc.VectorSubcoreMesh(core_axis_name="c", subcore_axis_name="s"),
        scratch_types=[
            pltpu.VMEM((CH,), jnp.int32),
            pltpu.VMEM((CH,), jnp.int32),
            pltpu.VMEM((CH, H), _f32),
            pltpu.VMEM((CH, H), _f32),
            pltpu.VMEM((8,), jnp.int32),
            pltpu.VMEM((8,), jnp.int32),
            pltpu.VMEM((8, H), _f32),
            pltpu.VMEM((8, H), _f32),
            pltpu.SemaphoreType.DMA,
        ],
    )(_gather_body)
    return f(*pqs, *srcs, *dsts)


def _mlp_body(ps_ref, qd_ref, w2_ref, w3_ref, w4_ref,
              b1_ref, b2_ref, b3_ref, b4_ref, m_ref):
    z = jnp.maximum(ps_ref[0] + qd_ref[0] + b1_ref[0], 0.0)
    z = jnp.maximum(
        jnp.dot(z, w2_ref[0], preferred_element_type=_f32) + b2_ref[0], 0.0)
    z = jnp.maximum(
        jnp.dot(z, w3_ref[0], preferred_element_type=_f32) + b3_ref[0], 0.0)
    m_ref[0] = jnp.dot(z, w4_ref[0], preferred_element_type=_f32) + b4_ref[0]


def _mlp(ps, qd, w2s, w3s, w4s, b1s, b2s, b3s, b4s):
    wspec = pl.BlockSpec((1, H, H), lambda et, t: (et, 0, 0))
    bspec = pl.BlockSpec((1, 1, H), lambda et, t: (et, 0, 0))
    espec = pl.BlockSpec((1, EB, H), lambda et, t: (et, t, 0))
    return pl.pallas_call(
        _mlp_body,
        grid=(NET, E // EB),
        in_specs=[espec, espec, wspec, wspec, wspec, bspec, bspec, bspec, bspec],
        out_specs=espec,
        out_shape=jax.ShapeDtypeStruct((NET, E, H), _f32),
    )(ps, qd, w2s, w3s, w4s, b1s, b2s, b3s, b4s)


def _scatter_body(m, d0, d1, d2, d3, s_out,
                  acc, idxv, rows, itd, rt, zv, sem):
    c = lax.axis_index("c")
    s = lax.axis_index("s")
    dsts = (d0, d1, d2, d3)

    for i in range(ZR):
        for k in range(H // 16):
            zv[i, pl.ds(k * 16, 16)] = jnp.zeros((16,), _f32)

    row0 = s * ROWS_PER_S
    for et in range(NET):
        def zchunk(r, carry):
            pltpu.sync_copy(zv, acc.at[pl.ds(row0 + r * ZR, ZR)])
            return carry
        lax.fori_loop(0, ROWS_PER_S // ZR, zchunk, 0)

        @pl.when(s == 15)
        def _():
            pltpu.sync_copy(zv, acc.at[pl.ds(16 * ROWS_PER_S, ROW_TAIL)])
        plsc.subcore_barrier()

        base_s = c * E_HALF + s * PER_S

        def chunk(j, carry, et=et):
            base = base_s + j * CH
            pltpu.sync_copy(dsts[et].at[pl.ds(base, CH)], idxv)
            pltpu.sync_copy(m.at[et, pl.ds(base, CH)], rows)
            pltpu.sync_copy(rows, acc.at[idxv], add=True)
            return carry
        lax.fori_loop(0, NCH, chunk, 0)

        @pl.when(s < SC_TAIL // 8)
        def _(et=et):
            tb = c * E_HALF + 16 * PER_S + s * 8
            pltpu.sync_copy(dsts[et].at[pl.ds(tb, 8)], itd)
            pltpu.sync_copy(m.at[et, pl.ds(tb, 8)], rt)
            pltpu.sync_copy(rt, acc.at[itd], add=True)
        plsc.subcore_barrier()

        pltpu.sync_copy(acc.at[pl.ds(row0, ROWS_PER_S)],
                        s_out.at[c, et, pl.ds(row0, ROWS_PER_S)])

        @pl.when(s == 15)
        def _(et=et):
            pltpu.sync_copy(acc.at[pl.ds(16 * ROWS_PER_S, ROW_TAIL)],
                            s_out.at[c, et, pl.ds(16 * ROWS_PER_S, ROW_TAIL)])
        plsc.subcore_barrier()


def _scatter(m, dsts):
    f = functools.partial(
        pl.kernel,
        out_type=jax.ShapeDtypeStruct((2, NET, N, H), _f32),
        mesh=plsc.VectorSubcoreMesh(core_axis_name="c", subcore_axis_name="s"),
        scratch_types=[
            pltpu.VMEM_SHARED((N, H), _f32),
            pltpu.VMEM((CH,), jnp.int32),
            pltpu.VMEM((CH, H), _f32),
            pltpu.VMEM((8,), jnp.int32),
            pltpu.VMEM((8, H), _f32),
            pltpu.VMEM((ZR, H), _f32),
            pltpu.SemaphoreType.DMA,
        ],
    )(_scatter_body)
    return f(m, *dsts)


def _gates1_body(x_ref, s_ref, wx_ref, wm_ref, h_ref, c_ref):
    x = x_ref[...]
    g = jnp.dot(x, wx_ref[...], preferred_element_type=_f32)
    for et in range(NET):
        sm = s_ref[0, et] + s_ref[1, et]
        g = g + jnp.dot(sm, wm_ref[et], preferred_element_type=_f32)
    i_g = jax.nn.sigmoid(g[:, 0:H])
    g_g = jnp.tanh(g[:, 2 * H:3 * H])
    o_g = jax.nn.sigmoid(g[:, 3 * H:4 * H])
    c_new = i_g * g_g
    c_ref[...] = c_new
    h_ref[...] = o_g * jnp.tanh(c_new)


def _gates1(x, s, wx, wm):
    nspec = pl.BlockSpec((NB, H), lambda t: (t, 0))
    return pl.pallas_call(
        _gates1_body,
        grid=(N // NB,),
        in_specs=[
            nspec,
            pl.BlockSpec((2, NET, NB, H), lambda t: (0, 0, t, 0)),
            pl.BlockSpec((H, 4 * H), lambda t: (0, 0)),
            pl.BlockSpec((NET, H, 4 * H), lambda t: (0, 0, 0)),
        ],
        out_specs=[nspec, nspec],
        out_shape=[jax.ShapeDtypeStruct((N, H), _f32)] * 2,
    )(x, s, wx, wm)


def _gates2_body(x_ref, s_ref, h_ref, c_ref, wx_ref, wm_ref, wh_ref, sw_ref,
                 lo_ref):
    x = x_ref[...]
    g = jnp.dot(x, wx_ref[...], preferred_element_type=_f32)
    g = g + jnp.dot(h_ref[...], wh_ref[...], preferred_element_type=_f32)
    for et in range(NET):
        sm = s_ref[0, et] + s_ref[1, et]
        g = g + jnp.dot(sm, wm_ref[et], preferred_element_type=_f32)
    i_g = jax.nn.sigmoid(g[:, 0:H])
    f_g = jax.nn.sigmoid(g[:, H:2 * H])
    g_g = jnp.tanh(g[:, 2 * H:3 * H])
    o_g = jax.nn.sigmoid(g[:, 3 * H:4 * H])
    c_new = f_g * c_ref[...] + i_g * g_g
    h_new = o_g * jnp.tanh(c_new)
    lo_ref[...] = jnp.dot(h_new, sw_ref[...], preferred_element_type=_f32)


def _gates2(x, s, h, cc, wx, wm, wh, swp):
    nspec = pl.BlockSpec((NB, H), lambda t: (t, 0))
    return pl.pallas_call(
        _gates2_body,
        grid=(N // NB,),
        in_specs=[
            nspec,
            pl.BlockSpec((2, NET, NB, H), lambda t: (0, 0, t, 0)),
            nspec,
            nspec,
            pl.BlockSpec((H, 4 * H), lambda t: (0, 0)),
            pl.BlockSpec((NET, H, 4 * H), lambda t: (0, 0, 0)),
            pl.BlockSpec((H, 4 * H), lambda t: (0, 0)),
            pl.BlockSpec((H, 8), lambda t: (0, 0)),
        ],
        out_specs=pl.BlockSpec((NB, 8), lambda t: (t, 0)),
        out_shape=jax.ShapeDtypeStruct((N, 8), _f32),
    )(x, s, h, cc, wx, wm, wh, swp)


def kernel(params, cell_idx, edge_index_intra_diff, edge_index_inter_diff,
           edge_index_intra_lt, edge_index_intra_gt):
    p = params
    ets = ('intra_diff', 'inter_diff', 'intra_lt', 'intra_gt')
    edges = (edge_index_intra_diff, edge_index_inter_diff,
             edge_index_intra_lt, edge_index_intra_gt)
    srcs = [e[0] for e in edges]
    dsts = [e[1] for e in edges]

    embp = jnp.zeros((8, H), _f32).at[:3, :].set(p['embed'])
    wpq = jnp.concatenate(
        [p['mlp_' + et]['W1'][:H] for et in ets]
        + [p['mlp_' + et]['W1'][H:] for et in ets], axis=1)
    w2s = jnp.stack([p['mlp_' + et]['W2'] for et in ets])
    w3s = jnp.stack([p['mlp_' + et]['W3'] for et in ets])
    w4s = jnp.stack([p['mlp_' + et]['W4'] for et in ets])
    b1s = jnp.stack([p['mlp_' + et]['b1'] for et in ets]).reshape(NET, 1, H)
    b2s = jnp.stack([p['mlp_' + et]['b2'] for et in ets]).reshape(NET, 1, H)
    b3s = jnp.stack([p['mlp_' + et]['b3'] for et in ets]).reshape(NET, 1, H)
    b4s = jnp.stack([p['mlp_' + et]['b4'] for et in ets]).reshape(NET, 1, H)
    wih = p['lstm_wih']
    wx = wih[:H]
    wm = wih[H:].reshape(NET, H, 4 * H)
    wh = p['lstm_whh']
    swp = jnp.zeros((H, 8), _f32).at[:, :1].set(p['score_w'])

    ci2 = jnp.broadcast_to(cell_idx.astype(jnp.int32)[:, None], (N, H))
    x = _embed(ci2, embp)

    h = x
    cc = None
    logits = None
    for step in range(STEPS):
        pqs = _pq(h, wpq)
        ps, qd = _gather(pqs, srcs, dsts)
        m = _mlp(ps, qd, w2s, w3s, w4s, b1s, b2s, b3s, b4s)
        s = _scatter(m, dsts)
        if step == 0:
            h, cc = _gates1(x, s, wx, wm)
        else:
            logits = _gates2(x, s, h, cc, wx, wm, wh, swp)
    return logits[:, 0]


# speedup vs baseline: 2.2339x; 1.4573x over previous
import functools

import jax
import jax.numpy as jnp
from jax import lax
from jax.experimental import pallas as pl
from jax.experimental.pallas import tpu as pltpu
from jax.experimental.pallas import tpu_sc as plsc

N = 10000
H = 128
E = 40000
NET = 4
STEPS = 2

NB = 1000
EB = 1000
CH = 96
NWORK = 32
NCH = 13
PER_W = NCH * CH
TAIL = E - NWORK * PER_W
E_HALF = E // 2
PER_S = NCH * CH
SC_TAIL = E_HALF - 16 * PER_S
ROWS_PER_S = 624
ROW_TAIL = N - 16 * ROWS_PER_S
ZR = 16

_f32 = jnp.float32


def _embed_body(ci_ref, emb_ref, x_ref):
    idx = ci_ref[...]
    acc = jnp.zeros((NB, H), _f32)
    for k in range(3):
        acc = acc + jnp.where(idx == k, 1.0, 0.0) * emb_ref[k][None, :]
    x_ref[...] = acc


def _embed(ci2, embp):
    return pl.pallas_call(
        _embed_body,
        grid=(N // NB,),
        in_specs=[
            pl.BlockSpec((NB, H), lambda t: (t, 0)),
            pl.BlockSpec((8, H), lambda t: (0, 0)),
        ],
        out_specs=pl.BlockSpec((NB, H), lambda t: (t, 0)),
        out_shape=jax.ShapeDtypeStruct((N, H), _f32),
    )(ci2, embp)


def _pq_body(h_ref, w_ref, *out_refs):
    acc = jnp.dot(h_ref[...], w_ref[...], preferred_element_type=_f32)
    for i in range(2 * NET):
        out_refs[i][...] = acc[:, i * H:(i + 1) * H]


def _pq(h, wpq):
    return pl.pallas_call(
        _pq_body,
        grid=(N // NB,),
        in_specs=[
            pl.BlockSpec((NB, H), lambda t: (t, 0)),
            pl.BlockSpec((H, 2 * NET * H), lambda t: (0, 0)),
        ],
        out_specs=[pl.BlockSpec((NB, H), lambda t: (t, 0))] * (2 * NET),
        out_shape=[jax.ShapeDtypeStruct((N, H), _f32)] * (2 * NET),
    )(h, wpq)


def _gather_body(p0, p1, p2, p3, q0, q1, q2, q3,
                 s0, s1, s2, s3, d0, d1, d2, d3,
                 z_out,
                 ib, rp, rq, its, itd, rtp, rtq,
                 isem, g0, g1, w0, w1, tsem):
    c = lax.axis_index("c")
    s = lax.axis_index("s")
    w = s * 2 + c
    base_w = w * PER_W
    ptabs = (p0, p1, p2, p3)
    qtabs = (q0, q1, q2, q3)
    srcs = (s0, s1, s2, s3)
    dsts = (d0, d1, d2, d3)
    gsems = (g0, g1)
    wsems = (w0, w1)

    idescs = []
    for et in range(NET):
        idescs.append(pltpu.async_copy(
            srcs[et].at[pl.ds(base_w, PER_W)],
            ib.at[pl.ds(et * PER_W, PER_W)], isem))
        idescs.append(pltpu.async_copy(
            dsts[et].at[pl.ds(base_w, PER_W)],
            ib.at[pl.ds((NET + et) * PER_W, PER_W)], isem))
    for dsc in idescs:
        dsc.wait()

    for et in range(NET):
        gd = [None] * NCH
        wd = [None] * NCH

        def issue(j, et=et, gd=gd):
            k = j % 2
            gd[j] = (
                pltpu.async_copy(
                    ptabs[et].at[ib.at[pl.ds(et * PER_W + j * CH, CH)]],
                    rp.at[k], gsems[k]),
                pltpu.async_copy(
                    qtabs[et].at[ib.at[pl.ds((NET + et) * PER_W + j * CH, CH)]],
                    rq.at[k], gsems[k]),
            )

        issue(0)
        for j in range(NCH):
            k = j % 2
            if j + 1 < NCH:
                if j >= 1:
                    wd[j - 1][0].wait()
                issue(j + 1)
            dp, dq = gd[j]
            dp.wait()
            dq.wait()

            def row(i, carry, k=k):
                for v in range(H // 16):
                    sl = pl.ds(v * 16, 16)
                    rp[k, i, sl] = rp[k, i, sl] + rq[k, i, sl]
                return carry
            lax.fori_loop(0, CH, row, 0)
            wd[j] = (pltpu.async_copy(
                rp.at[k], z_out.at[et, pl.ds(base_w + j * CH, CH)],
                wsems[k]),)
        wd[NCH - 2][0].wait()
        wd[NCH - 1][0].wait()

        @pl.when(w < TAIL // 8)
        def _(et=et):
            tb = NWORK * PER_W + w * 8
            pltpu.sync_copy(srcs[et].at[pl.ds(tb, 8)], its)
            pltpu.sync_copy(dsts[et].at[pl.ds(tb, 8)], itd)
            pltpu.async_copy(ptabs[et].at[its], rtp, tsem).wait()
            pltpu.async_copy(qtabs[et].at[itd], rtq, tsem).wait()

            def trow(i, carry):
                for v in range(H // 16):
                    sl = pl.ds(v * 16, 16)
                    rtp[i, sl] = rtp[i, sl] + rtq[i, sl]
                return carry
            lax.fori_loop(0, 8, trow, 0)
            pltpu.sync_copy(rtp, z_out.at[et, pl.ds(tb, 8)])


def _gather(pqs, srcs, dsts):
    f = functools.partial(
        pl.kernel,
        out_type=jax.ShapeDtypeStruct((NET, E, H), _f32),
        mesh=plsc.VectorSubcoreMesh(core_axis_name="c", subcore_axis_name="s"),
        scratch_types=[
            pltpu.VMEM((2 * NET * PER_W,), jnp.int32),
            pltpu.VMEM((2, CH, H), _f32),
            pltpu.VMEM((2, CH, H), _f32),
            pltpu.VMEM((8,), jnp.int32),
            pltpu.VMEM((8,), jnp.int32),
            pltpu.VMEM((8, H), _f32),
            pltpu.VMEM((8, H), _f32),
            pltpu.SemaphoreType.DMA,
            pltpu.SemaphoreType.DMA,
            pltpu.SemaphoreType.DMA,
            pltpu.SemaphoreType.DMA,
            pltpu.SemaphoreType.DMA,
            pltpu.SemaphoreType.DMA,
        ],
    )(_gather_body)
    return f(*pqs, *srcs, *dsts)


def _mlp_body(zs_ref, w2_ref, w3_ref, w4_ref,
              b1_ref, b2_ref, b3_ref, b4_ref, m_ref):
    z = jnp.maximum(zs_ref[0] + b1_ref[0], 0.0)
    z = jnp.maximum(
        jnp.dot(z, w2_ref[0], preferred_element_type=_f32) + b2_ref[0], 0.0)
    z = jnp.maximum(
        jnp.dot(z, w3_ref[0], preferred_element_type=_f32) + b3_ref[0], 0.0)
    m_ref[0] = jnp.dot(z, w4_ref[0], preferred_element_type=_f32) + b4_ref[0]


def _mlp(zs, w2s, w3s, w4s, b1s, b2s, b3s, b4s):
    wspec = pl.BlockSpec((1, H, H), lambda et, t: (et, 0, 0))
    bspec = pl.BlockSpec((1, 1, H), lambda et, t: (et, 0, 0))
    espec = pl.BlockSpec((1, EB, H), lambda et, t: (et, t, 0))
    return pl.pallas_call(
        _mlp_body,
        grid=(NET, E // EB),
        in_specs=[espec, wspec, wspec, wspec, bspec, bspec, bspec, bspec],
        out_specs=espec,
        out_shape=jax.ShapeDtypeStruct((NET, E, H), _f32),
    )(zs, w2s, w3s, w4s, b1s, b2s, b3s, b4s)


def _scatter_body(m, d0, d1, d2, d3, s_out,
                  acc, ix0, ix1, rows, itd, rt, zv,
                  i0, i1, r0, r1, a0, a1):
    c = lax.axis_index("c")
    s = lax.axis_index("s")
    dsts = (d0, d1, d2, d3)
    ixs = (ix0, ix1)
    isems = (i0, i1)
    rsems = (r0, r1)
    asems = (a0, a1)
    base_s = c * E_HALF + s * PER_S

    for i in range(ZR):
        for k in range(H // 16):
            zv[i, pl.ds(k * 16, 16)] = jnp.zeros((16,), _f32)

    row0 = s * ROWS_PER_S
    for et in range(NET):
        def zchunk(r, carry):
            pltpu.sync_copy(zv, acc.at[pl.ds(row0 + r * ZR, ZR)])
            return carry
        lax.fori_loop(0, ROWS_PER_S // ZR, zchunk, 0)

        @pl.when(s == 15)
        def _():
            pltpu.sync_copy(zv, acc.at[pl.ds(16 * ROWS_PER_S, ROW_TAIL)])
        plsc.subcore_barrier()

        rd = [None] * NCH
        xd = [None] * NCH
        ad = [None] * NCH

        def issue(j, et=et, rd=rd, xd=xd):
            k = j % 2
            xd[j] = pltpu.async_copy(
                dsts[et].at[pl.ds(base_s + j * CH, CH)], ixs[k], isems[k])
            rd[j] = pltpu.async_copy(
                m.at[et, pl.ds(base_s + j * CH, CH)], rows.at[k], rsems[k])

        issue(0)
        for j in range(NCH):
            k = j % 2
            if j + 1 < NCH:
                if j >= 1:
                    ad[j - 1].wait()
                issue(j + 1)
            rd[j].wait()
            xd[j].wait()
            ad[j] = pltpu.async_copy(
                rows.at[k], acc.at[ixs[k]], asems[k], add=True)
        ad[NCH - 2].wait()
        ad[NCH - 1].wait()

        @pl.when(s < SC_TAIL // 8)
        def _(et=et):
            tb = c * E_HALF + 16 * PER_S + s * 8
            pltpu.sync_copy(dsts[et].at[pl.ds(tb, 8)], itd)
            pltpu.sync_copy(m.at[et, pl.ds(tb, 8)], rt)
            pltpu.sync_copy(rt, acc.at[itd], add=True)
        plsc.subcore_barrier()

        pltpu.sync_copy(acc.at[pl.ds(row0, ROWS_PER_S)],
                        s_out.at[c, et, pl.ds(row0, ROWS_PER_S)])

        @pl.when(s == 15)
        def _(et=et):
            pltpu.sync_copy(acc.at[pl.ds(16 * ROWS_PER_S, ROW_TAIL)],
                            s_out.at[c, et, pl.ds(16 * ROWS_PER_S, ROW_TAIL)])
        plsc.subcore_barrier()


def _scatter(m, dsts):
    f = functools.partial(
        pl.kernel,
        out_type=jax.ShapeDtypeStruct((2, NET, N, H), _f32),
        mesh=plsc.VectorSubcoreMesh(core_axis_name="c", subcore_axis_name="s"),
        scratch_types=[
            pltpu.VMEM_SHARED((N, H), _f32),
            pltpu.VMEM((CH,), jnp.int32),
            pltpu.VMEM((CH,), jnp.int32),
            pltpu.VMEM((2, CH, H), _f32),
            pltpu.VMEM((8,), jnp.int32),
            pltpu.VMEM((8, H), _f32),
            pltpu.VMEM((ZR, H), _f32),
            pltpu.SemaphoreType.DMA,
            pltpu.SemaphoreType.DMA,
            pltpu.SemaphoreType.DMA,
            pltpu.SemaphoreType.DMA,
            pltpu.SemaphoreType.DMA,
            pltpu.SemaphoreType.DMA,
        ],
    )(_scatter_body)
    return f(m, *dsts)


def _gates1_body(x_ref, s_ref, wx_ref, wm_ref, h_ref, c_ref):
    x = x_ref[...]
    g = jnp.dot(x, wx_ref[...], preferred_element_type=_f32)
    for et in range(NET):
        sm = s_ref[0, et] + s_ref[1, et]
        g = g + jnp.dot(sm, wm_ref[et], preferred_element_type=_f32)
    i_g = jax.nn.sigmoid(g[:, 0:H])
    g_g = jnp.tanh(g[:, 2 * H:3 * H])
    o_g = jax.nn.sigmoid(g[:, 3 * H:4 * H])
    c_new = i_g * g_g
    c_ref[...] = c_new
    h_ref[...] = o_g * jnp.tanh(c_new)


def _gates1(x, s, wx, wm):
    nspec = pl.BlockSpec((NB, H), lambda t: (t, 0))
    return pl.pallas_call(
        _gates1_body,
        grid=(N // NB,),
        in_specs=[
            nspec,
            pl.BlockSpec((2, NET, NB, H), lambda t: (0, 0, t, 0)),
            pl.BlockSpec((H, 4 * H), lambda t: (0, 0)),
            pl.BlockSpec((NET, H, 4 * H), lambda t: (0, 0, 0)),
        ],
        out_specs=[nspec, nspec],
        out_shape=[jax.ShapeDtypeStruct((N, H), _f32)] * 2,
    )(x, s, wx, wm)


def _gates2_body(x_ref, s_ref, h_ref, c_ref, wx_ref, wm_ref, wh_ref, sw_ref,
                 lo_ref):
    x = x_ref[...]
    g = jnp.dot(x, wx_ref[...], preferred_element_type=_f32)
    g = g + jnp.dot(h_ref[...], wh_ref[...], preferred_element_type=_f32)
    for et in range(NET):
        sm = s_ref[0, et] + s_ref[1, et]
        g = g + jnp.dot(sm, wm_ref[et], preferred_element_type=_f32)
    i_g = jax.nn.sigmoid(g[:, 0:H])
    f_g = jax.nn.sigmoid(g[:, H:2 * H])
    g_g = jnp.tanh(g[:, 2 * H:3 * H])
    o_g = jax.nn.sigmoid(g[:, 3 * H:4 * H])
    c_new = f_g * c_ref[...] + i_g * g_g
    h_new = o_g * jnp.tanh(c_new)
    lo_ref[...] = jnp.dot(h_new, sw_ref[...], preferred_element_type=_f32)


def _gates2(x, s, h, cc, wx, wm, wh, swp):
    nspec = pl.BlockSpec((NB, H), lambda t: (t, 0))
    return pl.pallas_call(
        _gates2_body,
        grid=(N // NB,),
        in_specs=[
            nspec,
            pl.BlockSpec((2, NET, NB, H), lambda t: (0, 0, t, 0)),
            nspec,
            nspec,
            pl.BlockSpec((H, 4 * H), lambda t: (0, 0)),
            pl.BlockSpec((NET, H, 4 * H), lambda t: (0, 0, 0)),
            pl.BlockSpec((H, 4 * H), lambda t: (0, 0)),
            pl.BlockSpec((H, 8), lambda t: (0, 0)),
        ],
        out_specs=pl.BlockSpec((NB, 8), lambda t: (t, 0)),
        out_shape=jax.ShapeDtypeStruct((N, 8), _f32),
    )(x, s, h, cc, wx, wm, wh, swp)


def kernel(params, cell_idx, edge_index_intra_diff, edge_index_inter_diff,
           edge_index_intra_lt, edge_index_intra_gt):
    p = params
    ets = ('intra_diff', 'inter_diff', 'intra_lt', 'intra_gt')
    edges = (edge_index_intra_diff, edge_index_inter_diff,
             edge_index_intra_lt, edge_index_intra_gt)
    srcs = [e[0] for e in edges]
    dsts = [e[1] for e in edges]

    embp = jnp.zeros((8, H), _f32).at[:3, :].set(p['embed'])
    wpq = jnp.concatenate(
        [p['mlp_' + et]['W1'][:H] for et in ets]
        + [p['mlp_' + et]['W1'][H:] for et in ets], axis=1)
    w2s = jnp.stack([p['mlp_' + et]['W2'] for et in ets])
    w3s = jnp.stack([p['mlp_' + et]['W3'] for et in ets])
    w4s = jnp.stack([p['mlp_' + et]['W4'] for et in ets])
    b1s = jnp.stack([p['mlp_' + et]['b1'] for et in ets]).reshape(NET, 1, H)
    b2s = jnp.stack([p['mlp_' + et]['b2'] for et in ets]).reshape(NET, 1, H)
    b3s = jnp.stack([p['mlp_' + et]['b3'] for et in ets]).reshape(NET, 1, H)
    b4s = jnp.stack([p['mlp_' + et]['b4'] for et in ets]).reshape(NET, 1, H)
    wih = p['lstm_wih']
    wx = wih[:H]
    wm = wih[H:].reshape(NET, H, 4 * H)
    wh = p['lstm_whh']
    swp = jnp.zeros((H, 8), _f32).at[:, :1].set(p['score_w'])

    ci2 = jnp.broadcast_to(cell_idx.astype(jnp.int32)[:, None], (N, H))
    x = _embed(ci2, embp)

    h = x
    cc = None
    logits = None
    for step in range(STEPS):
        pqs = _pq(h, wpq)
        zs = _gather(pqs, srcs, dsts)
        m = _mlp(zs, w2s, w3s, w4s, b1s, b2s, b3s, b4s)
        s = _scatter(m, dsts)
        if step == 0:
            h, cc = _gates1(x, s, wx, wm)
        else:
            logits = _gates2(x, s, h, cc, wx, wm, wh, swp)
    return logits[:, 0]


# speedup vs baseline: 2.2404x; 1.0029x over previous
import functools

import jax
import jax.numpy as jnp
from jax import lax
from jax.experimental import pallas as pl
from jax.experimental.pallas import tpu as pltpu
from jax.experimental.pallas import tpu_sc as plsc

N = 10000
H = 128
E = 40000
NET = 4
STEPS = 2

NB = 1000
EB = 1000
CH = 96
NWORK = 32
NCH = 13
PER_W = NCH * CH
TAIL = E - NWORK * PER_W
E_HALF = E // 2
PER_S = NCH * CH
SC_TAIL = E_HALF - 16 * PER_S
ROWS_PER_S = 624
ROW_TAIL = N - 16 * ROWS_PER_S
ZR = 16

_f32 = jnp.float32


def _embed_body(ci_ref, emb_ref, x_ref):
    idx = ci_ref[...]
    acc = jnp.zeros((NB, H), _f32)
    for k in range(3):
        acc = acc + jnp.where(idx == k, 1.0, 0.0) * emb_ref[k][None, :]
    x_ref[...] = acc


def _embed(ci2, embp):
    return pl.pallas_call(
        _embed_body,
        grid=(N // NB,),
        in_specs=[
            pl.BlockSpec((NB, H), lambda t: (t, 0)),
            pl.BlockSpec((8, H), lambda t: (0, 0)),
        ],
        out_specs=pl.BlockSpec((NB, H), lambda t: (t, 0)),
        out_shape=jax.ShapeDtypeStruct((N, H), _f32),
    )(ci2, embp)


def _pq_body(h_ref, w_ref, *out_refs):
    acc = jnp.dot(h_ref[...].astype(jnp.bfloat16), w_ref[...],
                  preferred_element_type=_f32)
    for i in range(2 * NET):
        out_refs[i][...] = acc[:, i * H:(i + 1) * H]


def _pq(h, wpq):
    return pl.pallas_call(
        _pq_body,
        grid=(N // NB,),
        in_specs=[
            pl.BlockSpec((NB, H), lambda t: (t, 0)),
            pl.BlockSpec((H, 2 * NET * H), lambda t: (0, 0)),
        ],
        out_specs=[pl.BlockSpec((NB, H), lambda t: (t, 0))] * (2 * NET),
        out_shape=[jax.ShapeDtypeStruct((N, H), _f32)] * (2 * NET),
    )(h, wpq.astype(jnp.bfloat16))


def _gather_body(p0, p1, p2, p3, q0, q1, q2, q3,
                 s0, s1, s2, s3, d0, d1, d2, d3,
                 z_out,
                 ib, rp, rq, its, itd, rtp, rtq,
                 isem, g0, g1, w0, w1, tsem):
    c = lax.axis_index("c")
    s = lax.axis_index("s")
    w = s * 2 + c
    base_w = w * PER_W
    ptabs = (p0, p1, p2, p3)
    qtabs = (q0, q1, q2, q3)
    srcs = (s0, s1, s2, s3)
    dsts = (d0, d1, d2, d3)
    gsems = (g0, g1)
    wsems = (w0, w1)

    idescs = []
    for et in range(NET):
        idescs.append(pltpu.async_copy(
            srcs[et].at[pl.ds(base_w, PER_W)],
            ib.at[pl.ds(et * PER_W, PER_W)], isem))
        idescs.append(pltpu.async_copy(
            dsts[et].at[pl.ds(base_w, PER_W)],
            ib.at[pl.ds((NET + et) * PER_W, PER_W)], isem))
    for dsc in idescs:
        dsc.wait()

    for et in range(NET):
        gd = [None] * NCH
        wd = [None] * NCH

        def issue(j, et=et, gd=gd):
            k = j % 2
            gd[j] = (
                pltpu.async_copy(
                    ptabs[et].at[ib.at[pl.ds(et * PER_W + j * CH, CH)]],
                    rp.at[k], gsems[k]),
                pltpu.async_copy(
                    qtabs[et].at[ib.at[pl.ds((NET + et) * PER_W + j * CH, CH)]],
                    rq.at[k], gsems[k]),
            )

        issue(0)
        for j in range(NCH):
            k = j % 2
            if j + 1 < NCH:
                if j >= 1:
                    wd[j - 1][0].wait()
                issue(j + 1)
            dp, dq = gd[j]
            dp.wait()
            dq.wait()

            def row(i, carry, k=k):
                for v in range(H // 16):
                    sl = pl.ds(v * 16, 16)
                    rp[k, i, sl] = rp[k, i, sl] + rq[k, i, sl]
                return carry
            lax.fori_loop(0, CH, row, 0)
            wd[j] = (pltpu.async_copy(
                rp.at[k], z_out.at[et, pl.ds(base_w + j * CH, CH)],
                wsems[k]),)
        wd[NCH - 2][0].wait()
        wd[NCH - 1][0].wait()

        @pl.when(w < TAIL // 8)
        def _(et=et):
            tb = NWORK * PER_W + w * 8
            pltpu.sync_copy(srcs[et].at[pl.ds(tb, 8)], its)
            pltpu.sync_copy(dsts[et].at[pl.ds(tb, 8)], itd)
            pltpu.async_copy(ptabs[et].at[its], rtp, tsem).wait()
            pltpu.async_copy(qtabs[et].at[itd], rtq, tsem).wait()

            def trow(i, carry):
                for v in range(H // 16):
                    sl = pl.ds(v * 16, 16)
                    rtp[i, sl] = rtp[i, sl] + rtq[i, sl]
                return carry
            lax.fori_loop(0, 8, trow, 0)
            pltpu.sync_copy(rtp, z_out.at[et, pl.ds(tb, 8)])


def _gather(pqs, srcs, dsts):
    f = functools.partial(
        pl.kernel,
        out_type=jax.ShapeDtypeStruct((NET, E, H), _f32),
        mesh=plsc.VectorSubcoreMesh(core_axis_name="c", subcore_axis_name="s"),
        scratch_types=[
            pltpu.VMEM((2 * NET * PER_W,), jnp.int32),
            pltpu.VMEM((2, CH, H), _f32),
            pltpu.VMEM((2, CH, H), _f32),
            pltpu.VMEM((8,), jnp.int32),
            pltpu.VMEM((8,), jnp.int32),
            pltpu.VMEM((8, H), _f32),
            pltpu.VMEM((8, H), _f32),
            pltpu.SemaphoreType.DMA,
            pltpu.SemaphoreType.DMA,
            pltpu.SemaphoreType.DMA,
            pltpu.SemaphoreType.DMA,
            pltpu.SemaphoreType.DMA,
            pltpu.SemaphoreType.DMA,
        ],
    )(_gather_body)
    return f(*pqs, *srcs, *dsts)


def _mlp_body(zs_ref, w2_ref, w3_ref, w4_ref,
              b1_ref, b2_ref, b3_ref, b4_ref, m_ref):
    z = jnp.maximum(zs_ref[0] + b1_ref[0], 0.0).astype(jnp.bfloat16)
    z = jnp.maximum(
        jnp.dot(z, w2_ref[0], preferred_element_type=_f32) + b2_ref[0],
        0.0).astype(jnp.bfloat16)
    z = jnp.maximum(
        jnp.dot(z, w3_ref[0], preferred_element_type=_f32) + b3_ref[0],
        0.0).astype(jnp.bfloat16)
    m_ref[0] = jnp.dot(z, w4_ref[0], preferred_element_type=_f32) + b4_ref[0]


def _mlp(zs, w2s, w3s, w4s, b1s, b2s, b3s, b4s):
    wspec = pl.BlockSpec((1, H, H), lambda et, t: (et, 0, 0))
    bspec = pl.BlockSpec((1, 1, H), lambda et, t: (et, 0, 0))
    espec = pl.BlockSpec((1, EB, H), lambda et, t: (et, t, 0))
    return pl.pallas_call(
        _mlp_body,
        grid=(NET, E // EB),
        in_specs=[espec, wspec, wspec, wspec, bspec, bspec, bspec, bspec],
        out_specs=espec,
        out_shape=jax.ShapeDtypeStruct((NET, E, H), _f32),
    )(zs, w2s.astype(jnp.bfloat16), w3s.astype(jnp.bfloat16),
      w4s.astype(jnp.bfloat16), b1s, b2s, b3s, b4s)


def _scatter_body(m, d0, d1, d2, d3, s_out,
                  acc, ix0, ix1, rows, itd, rt, zv,
                  i0, i1, r0, r1, a0, a1):
    c = lax.axis_index("c")
    s = lax.axis_index("s")
    dsts = (d0, d1, d2, d3)
    ixs = (ix0, ix1)
    isems = (i0, i1)
    rsems = (r0, r1)
    asems = (a0, a1)
    base_s = c * E_HALF + s * PER_S

    for i in range(ZR):
        for k in range(H // 16):
            zv[i, pl.ds(k * 16, 16)] = jnp.zeros((16,), _f32)

    row0 = s * ROWS_PER_S
    for et in range(NET):
        def zchunk(r, carry):
            pltpu.sync_copy(zv, acc.at[pl.ds(row0 + r * ZR, ZR)])
            return carry
        lax.fori_loop(0, ROWS_PER_S // ZR, zchunk, 0)

        @pl.when(s == 15)
        def _():
            pltpu.sync_copy(zv, acc.at[pl.ds(16 * ROWS_PER_S, ROW_TAIL)])
        plsc.subcore_barrier()

        rd = [None] * NCH
        xd = [None] * NCH
        ad = [None] * NCH

        def issue(j, et=et, rd=rd, xd=xd):
            k = j % 2
            xd[j] = pltpu.async_copy(
                dsts[et].at[pl.ds(base_s + j * CH, CH)], ixs[k], isems[k])
            rd[j] = pltpu.async_copy(
                m.at[et, pl.ds(base_s + j * CH, CH)], rows.at[k], rsems[k])

        issue(0)
        for j in range(NCH):
            k = j % 2
            if j + 1 < NCH:
                if j >= 1:
                    ad[j - 1].wait()
                issue(j + 1)
            rd[j].wait()
            xd[j].wait()
            ad[j] = pltpu.async_copy(
                rows.at[k], acc.at[ixs[k]], asems[k], add=True)
        ad[NCH - 2].wait()
        ad[NCH - 1].wait()

        @pl.when(s < SC_TAIL // 8)
        def _(et=et):
            tb = c * E_HALF + 16 * PER_S + s * 8
            pltpu.sync_copy(dsts[et].at[pl.ds(tb, 8)], itd)
            pltpu.sync_copy(m.at[et, pl.ds(tb, 8)], rt)
            pltpu.sync_copy(rt, acc.at[itd], add=True)
        plsc.subcore_barrier()

        pltpu.sync_copy(acc.at[pl.ds(row0, ROWS_PER_S)],
                        s_out.at[c, et, pl.ds(row0, ROWS_PER_S)])

        @pl.when(s == 15)
        def _(et=et):
            pltpu.sync_copy(acc.at[pl.ds(16 * ROWS_PER_S, ROW_TAIL)],
                            s_out.at[c, et, pl.ds(16 * ROWS_PER_S, ROW_TAIL)])
        plsc.subcore_barrier()


def _scatter(m, dsts):
    f = functools.partial(
        pl.kernel,
        out_type=jax.ShapeDtypeStruct((2, NET, N, H), _f32),
        mesh=plsc.VectorSubcoreMesh(core_axis_name="c", subcore_axis_name="s"),
        scratch_types=[
            pltpu.VMEM_SHARED((N, H), _f32),
            pltpu.VMEM((CH,), jnp.int32),
            pltpu.VMEM((CH,), jnp.int32),
            pltpu.VMEM((2, CH, H), _f32),
            pltpu.VMEM((8,), jnp.int32),
            pltpu.VMEM((8, H), _f32),
            pltpu.VMEM((ZR, H), _f32),
            pltpu.SemaphoreType.DMA,
            pltpu.SemaphoreType.DMA,
            pltpu.SemaphoreType.DMA,
            pltpu.SemaphoreType.DMA,
            pltpu.SemaphoreType.DMA,
            pltpu.SemaphoreType.DMA,
        ],
    )(_scatter_body)
    return f(m, *dsts)


def _gates1_body(x_ref, s_ref, wx_ref, wm_ref, h_ref, c_ref):
    x = x_ref[...].astype(jnp.bfloat16)
    g = jnp.dot(x, wx_ref[...], preferred_element_type=_f32)
    for et in range(NET):
        sm = (s_ref[0, et] + s_ref[1, et]).astype(jnp.bfloat16)
        g = g + jnp.dot(sm, wm_ref[et], preferred_element_type=_f32)
    i_g = jax.nn.sigmoid(g[:, 0:H])
    g_g = jnp.tanh(g[:, 2 * H:3 * H])
    o_g = jax.nn.sigmoid(g[:, 3 * H:4 * H])
    c_new = i_g * g_g
    c_ref[...] = c_new
    h_ref[...] = o_g * jnp.tanh(c_new)


def _gates1(x, s, wx, wm):
    nspec = pl.BlockSpec((NB, H), lambda t: (t, 0))
    return pl.pallas_call(
        _gates1_body,
        grid=(N // NB,),
        in_specs=[
            nspec,
            pl.BlockSpec((2, NET, NB, H), lambda t: (0, 0, t, 0)),
            pl.BlockSpec((H, 4 * H), lambda t: (0, 0)),
            pl.BlockSpec((NET, H, 4 * H), lambda t: (0, 0, 0)),
        ],
        out_specs=[nspec, nspec],
        out_shape=[jax.ShapeDtypeStruct((N, H), _f32)] * 2,
    )(x, s, wx.astype(jnp.bfloat16), wm.astype(jnp.bfloat16))


def _gates2_body(x_ref, s_ref, h_ref, c_ref, wx_ref, wm_ref, wh_ref, sw_ref,
                 lo_ref):
    x = x_ref[...].astype(jnp.bfloat16)
    g = jnp.dot(x, wx_ref[...], preferred_element_type=_f32)
    g = g + jnp.dot(h_ref[...].astype(jnp.bfloat16), wh_ref[...],
                    preferred_element_type=_f32)
    for et in range(NET):
        sm = (s_ref[0, et] + s_ref[1, et]).astype(jnp.bfloat16)
        g = g + jnp.dot(sm, wm_ref[et], preferred_element_type=_f32)
    i_g = jax.nn.sigmoid(g[:, 0:H])
    f_g = jax.nn.sigmoid(g[:, H:2 * H])
    g_g = jnp.tanh(g[:, 2 * H:3 * H])
    o_g = jax.nn.sigmoid(g[:, 3 * H:4 * H])
    c_new = f_g * c_ref[...] + i_g * g_g
    h_new = o_g * jnp.tanh(c_new)
    lo_ref[...] = jnp.dot(h_new.astype(jnp.bfloat16), sw_ref[...],
                          preferred_element_type=_f32)


def _gates2(x, s, h, cc, wx, wm, wh, swp):
    nspec = pl.BlockSpec((NB, H), lambda t: (t, 0))
    return pl.pallas_call(
        _gates2_body,
        grid=(N // NB,),
        in_specs=[
            nspec,
            pl.BlockSpec((2, NET, NB, H), lambda t: (0, 0, t, 0)),
            nspec,
            nspec,
            pl.BlockSpec((H, 4 * H), lambda t: (0, 0)),
            pl.BlockSpec((NET, H, 4 * H), lambda t: (0, 0, 0)),
            pl.BlockSpec((H, 4 * H), lambda t: (0, 0)),
            pl.BlockSpec((H, 8), lambda t: (0, 0)),
        ],
        out_specs=pl.BlockSpec((NB, 8), lambda t: (t, 0)),
        out_shape=jax.ShapeDtypeStruct((N, 8), _f32),
    )(x, s, h, cc, wx.astype(jnp.bfloat16), wm.astype(jnp.bfloat16),
      wh.astype(jnp.bfloat16), swp.astype(jnp.bfloat16))


def kernel(params, cell_idx, edge_index_intra_diff, edge_index_inter_diff,
           edge_index_intra_lt, edge_index_intra_gt):
    p = params
    ets = ('intra_diff', 'inter_diff', 'intra_lt', 'intra_gt')
    edges = (edge_index_intra_diff, edge_index_inter_diff,
             edge_index_intra_lt, edge_index_intra_gt)
    srcs = [e[0] for e in edges]
    dsts = [e[1] for e in edges]

    embp = jnp.zeros((8, H), _f32).at[:3, :].set(p['embed'])
    wpq = jnp.concatenate(
        [p['mlp_' + et]['W1'][:H] for et in ets]
        + [p['mlp_' + et]['W1'][H:] for et in ets], axis=1)
    w2s = jnp.stack([p['mlp_' + et]['W2'] for et in ets])
    w3s = jnp.stack([p['mlp_' + et]['W3'] for et in ets])
    w4s = jnp.stack([p['mlp_' + et]['W4'] for et in ets])
    b1s = jnp.stack([p['mlp_' + et]['b1'] for et in ets]).reshape(NET, 1, H)
    b2s = jnp.stack([p['mlp_' + et]['b2'] for et in ets]).reshape(NET, 1, H)
    b3s = jnp.stack([p['mlp_' + et]['b3'] for et in ets]).reshape(NET, 1, H)
    b4s = jnp.stack([p['mlp_' + et]['b4'] for et in ets]).reshape(NET, 1, H)
    wih = p['lstm_wih']
    wx = wih[:H]
    wm = wih[H:].reshape(NET, H, 4 * H)
    wh = p['lstm_whh']
    swp = jnp.zeros((H, 8), _f32).at[:, :1].set(p['score_w'])

    ci2 = jnp.broadcast_to(cell_idx.astype(jnp.int32)[:, None], (N, H))
    x = _embed(ci2, embp)

    h = x
    cc = None
    logits = None
    for step in range(STEPS):
        pqs = _pq(h, wpq)
        zs = _gather(pqs, srcs, dsts)
        m = _mlp(zs, w2s, w3s, w4s, b1s, b2s, b3s, b4s)
        s = _scatter(m, dsts)
        if step == 0:
            h, cc = _gates1(x, s, wx, wm)
        else:
            logits = _gates2(x, s, h, cc, wx, wm, wh, swp)
    return logits[:, 0]


# speedup vs baseline: 2.4945x; 1.1134x over previous
import functools

import jax
import jax.numpy as jnp
from jax import lax
from jax.experimental import pallas as pl
from jax.experimental.pallas import tpu as pltpu
from jax.experimental.pallas import tpu_sc as plsc

N = 10000
H = 128
E = 40000
NET = 4
STEPS = 2

NB = 1000
EB = 2000
CH = 96
NWORK = 32
NCH = 13
PER_W = NCH * CH
TAIL = E - NWORK * PER_W
E_HALF = E // 2
PER_S = NCH * CH
SC_TAIL = E_HALF - 16 * PER_S
ROWS_PER_S = 624
ROW_TAIL = N - 16 * ROWS_PER_S
ZR = 16

_f32 = jnp.float32


def _embed_body(ci_ref, emb_ref, x_ref):
    idx = ci_ref[...]
    acc = jnp.zeros((NB, H), _f32)
    for k in range(3):
        acc = acc + jnp.where(idx == k, 1.0, 0.0) * emb_ref[k][None, :]
    x_ref[...] = acc


def _embed(ci2, embp):
    return pl.pallas_call(
        _embed_body,
        grid=(N // NB,),
        in_specs=[
            pl.BlockSpec((NB, H), lambda t: (t, 0)),
            pl.BlockSpec((8, H), lambda t: (0, 0)),
        ],
        out_specs=pl.BlockSpec((NB, H), lambda t: (t, 0)),
        out_shape=jax.ShapeDtypeStruct((N, H), _f32),
    )(ci2, embp)


def _pq_body(h_ref, w_ref, *out_refs):
    acc = jnp.dot(h_ref[...].astype(jnp.bfloat16), w_ref[...],
                  preferred_element_type=_f32)
    for i in range(2 * NET):
        lo = lax.bitcast_convert_type(acc[:, i * H:i * H + H // 2],
                                      jnp.uint32)
        hi = lax.bitcast_convert_type(acc[:, i * H + H // 2:(i + 1) * H],
                                      jnp.uint32)
        word = (((lo + jnp.uint32(0x8000)) >> 16)
                | ((hi + jnp.uint32(0x8000)) & jnp.uint32(0xFFFF0000)))
        out_refs[i][...] = lax.bitcast_convert_type(word, jnp.int32)


def _pq(h, wpq):
    return pl.pallas_call(
        _pq_body,
        grid=(N // NB,),
        in_specs=[
            pl.BlockSpec((NB, H), lambda t: (t, 0)),
            pl.BlockSpec((H, 2 * NET * H), lambda t: (0, 0)),
        ],
        out_specs=[pl.BlockSpec((NB, H // 2), lambda t: (t, 0))] * (2 * NET),
        out_shape=[jax.ShapeDtypeStruct((N, H // 2), jnp.int32)] * (2 * NET),
    )(h, wpq.astype(jnp.bfloat16))


def _gather_body(p0, p1, p2, p3, q0, q1, q2, q3,
                 s0, s1, s2, s3, d0, d1, d2, d3,
                 zp_out, zq_out,
                 ib, rp, rq, its, itd, rtp, rtq,
                 isem, g0, g1, w0, w1, tsem):
    c = lax.axis_index("c")
    s = lax.axis_index("s")
    w = s * 2 + c
    base_w = w * PER_W
    ptabs = (p0, p1, p2, p3)
    qtabs = (q0, q1, q2, q3)
    srcs = (s0, s1, s2, s3)
    dsts = (d0, d1, d2, d3)
    gsems = (g0, g1)
    wsems = (w0, w1)

    idescs = []
    for et in range(NET):
        idescs.append(pltpu.async_copy(
            srcs[et].at[pl.ds(base_w, PER_W)],
            ib.at[pl.ds(et * PER_W, PER_W)], isem))
        idescs.append(pltpu.async_copy(
            dsts[et].at[pl.ds(base_w, PER_W)],
            ib.at[pl.ds((NET + et) * PER_W, PER_W)], isem))
    for dsc in idescs:
        dsc.wait()

    for et in range(NET):
        gd = [None] * NCH
        wd = [None] * NCH

        def issue(j, et=et, gd=gd):
            k = j % 2
            gd[j] = (
                pltpu.async_copy(
                    ptabs[et].at[ib.at[pl.ds(et * PER_W + j * CH, CH)]],
                    rp.at[k], gsems[k]),
                pltpu.async_copy(
                    qtabs[et].at[ib.at[pl.ds((NET + et) * PER_W + j * CH, CH)]],
                    rq.at[k], gsems[k]),
            )

        issue(0)
        for j in range(NCH):
            k = j % 2
            if j + 1 < NCH:
                if j >= 1:
                    wd[j - 1][0].wait()
                    wd[j - 1][1].wait()
                issue(j + 1)
            dp, dq = gd[j]
            dp.wait()
            dq.wait()
            wd[j] = (
                pltpu.async_copy(
                    rp.at[k],
                    zp_out.at[et, pl.ds(base_w + j * CH, CH)], wsems[k]),
                pltpu.async_copy(
                    rq.at[k],
                    zq_out.at[et, pl.ds(base_w + j * CH, CH)], wsems[k]),
            )
        for j in (NCH - 2, NCH - 1):
            wd[j][0].wait()
            wd[j][1].wait()

        @pl.when(w < TAIL // 16)
        def _(et=et):
            tb = NWORK * PER_W + w * 16
            pltpu.sync_copy(srcs[et].at[pl.ds(tb, 16)], its)
            pltpu.sync_copy(dsts[et].at[pl.ds(tb, 16)], itd)
            pltpu.async_copy(ptabs[et].at[its], rtp, tsem).wait()
            pltpu.async_copy(qtabs[et].at[itd], rtq, tsem).wait()
            pltpu.sync_copy(rtp, zp_out.at[et, pl.ds(tb, 16)])
            pltpu.sync_copy(rtq, zq_out.at[et, pl.ds(tb, 16)])


def _gather(pqs, srcs, dsts):
    f = functools.partial(
        pl.kernel,
        out_type=[jax.ShapeDtypeStruct((NET, E, 1, H // 2), jnp.int32)] * 2,
        mesh=plsc.VectorSubcoreMesh(core_axis_name="c", subcore_axis_name="s"),
        scratch_types=[
            pltpu.VMEM((2 * NET * PER_W,), jnp.int32),
            pltpu.VMEM((2, CH, 1, H // 2), jnp.int32),
            pltpu.VMEM((2, CH, 1, H // 2), jnp.int32),
            pltpu.VMEM((16,), jnp.int32),
            pltpu.VMEM((16,), jnp.int32),
            pltpu.VMEM((16, 1, H // 2), jnp.int32),
            pltpu.VMEM((16, 1, H // 2), jnp.int32),
            pltpu.SemaphoreType.DMA,
            pltpu.SemaphoreType.DMA,
            pltpu.SemaphoreType.DMA,
            pltpu.SemaphoreType.DMA,
            pltpu.SemaphoreType.DMA,
            pltpu.SemaphoreType.DMA,
        ],
    )(_gather_body)
    return f(*pqs, *srcs, *dsts)


def _unpack_z(z_ref):
    u = lax.bitcast_convert_type(z_ref[0], jnp.uint32)
    lo = lax.bitcast_convert_type(u << 16, _f32)
    hi = lax.bitcast_convert_type(u & jnp.uint32(0xFFFF0000), _f32)
    return lo, hi


def _mlp_body(zp_ref, zq_ref, w2_ref, w3_ref, w4_ref,
              b1_ref, b2_ref, b3_ref, b4_ref, m_ref):
    plo, phi = _unpack_z(zp_ref)
    qlo, qhi = _unpack_z(zq_ref)
    zf = jnp.concatenate([plo + qlo, phi + qhi], axis=1)
    z = jnp.maximum(zf + b1_ref[0], 0.0).astype(jnp.bfloat16)
    z = jnp.maximum(
        jnp.dot(z, w2_ref[0], preferred_element_type=_f32) + b2_ref[0],
        0.0).astype(jnp.bfloat16)
    z = jnp.maximum(
        jnp.dot(z, w3_ref[0], preferred_element_type=_f32) + b3_ref[0],
        0.0).astype(jnp.bfloat16)
    m_ref[0] = jnp.dot(z, w4_ref[0], preferred_element_type=_f32) + b4_ref[0]


def _mlp(zp, zq, w2s, w3s, w4s, b1s, b2s, b3s, b4s):
    wspec = pl.BlockSpec((1, H, H), lambda et, t: (et, 0, 0))
    bspec = pl.BlockSpec((1, 1, H), lambda et, t: (et, 0, 0))
    zspec = pl.BlockSpec((1, EB, H // 2), lambda et, t: (et, t, 0))
    espec = pl.BlockSpec((1, EB, H), lambda et, t: (et, t, 0))
    return pl.pallas_call(
        _mlp_body,
        grid=(NET, E // EB),
        in_specs=[zspec, zspec, wspec, wspec, wspec,
                  bspec, bspec, bspec, bspec],
        out_specs=espec,
        out_shape=jax.ShapeDtypeStruct((NET, E, H), _f32),
    )(zp, zq, w2s.astype(jnp.bfloat16), w3s.astype(jnp.bfloat16),
      w4s.astype(jnp.bfloat16), b1s, b2s, b3s, b4s)


def _scatter_body(m, d0, d1, d2, d3, s_out,
                  acc, ix0, ix1, rows, itd, rt, zv,
                  i0, i1, r0, r1, a0, a1):
    c = lax.axis_index("c")
    s = lax.axis_index("s")
    dsts = (d0, d1, d2, d3)
    ixs = (ix0, ix1)
    isems = (i0, i1)
    rsems = (r0, r1)
    asems = (a0, a1)
    base_s = c * E_HALF + s * PER_S

    for i in range(ZR):
        for k in range(H // 16):
            zv[i, pl.ds(k * 16, 16)] = jnp.zeros((16,), _f32)

    row0 = s * ROWS_PER_S
    for et in range(NET):
        def zchunk(r, carry):
            pltpu.sync_copy(zv, acc.at[pl.ds(row0 + r * ZR, ZR)])
            return carry
        lax.fori_loop(0, ROWS_PER_S // ZR, zchunk, 0)

        @pl.when(s == 15)
        def _():
            pltpu.sync_copy(zv, acc.at[pl.ds(16 * ROWS_PER_S, ROW_TAIL)])
        plsc.subcore_barrier()

        rd = [None] * NCH
        xd = [None] * NCH
        ad = [None] * NCH

        def issue(j, et=et, rd=rd, xd=xd):
            k = j % 2
            xd[j] = pltpu.async_copy(
                dsts[et].at[pl.ds(base_s + j * CH, CH)], ixs[k], isems[k])
            rd[j] = pltpu.async_copy(
                m.at[et, pl.ds(base_s + j * CH, CH)], rows.at[k], rsems[k])

        issue(0)
        for j in range(NCH):
            k = j % 2
            if j + 1 < NCH:
                if j >= 1:
                    ad[j - 1].wait()
                issue(j + 1)
            rd[j].wait()
            xd[j].wait()
            ad[j] = pltpu.async_copy(
                rows.at[k], acc.at[ixs[k]], asems[k], add=True)
        ad[NCH - 2].wait()
        ad[NCH - 1].wait()

        @pl.when(s < SC_TAIL // 8)
        def _(et=et):
            tb = c * E_HALF + 16 * PER_S + s * 8
            pltpu.sync_copy(dsts[et].at[pl.ds(tb, 8)], itd)
            pltpu.sync_copy(m.at[et, pl.ds(tb, 8)], rt)
            pltpu.sync_copy(rt, acc.at[itd], add=True)
        plsc.subcore_barrier()

        pltpu.sync_copy(acc.at[pl.ds(row0, ROWS_PER_S)],
                        s_out.at[c, et, pl.ds(row0, ROWS_PER_S)])

        @pl.when(s == 15)
        def _(et=et):
            pltpu.sync_copy(acc.at[pl.ds(16 * ROWS_PER_S, ROW_TAIL)],
                            s_out.at[c, et, pl.ds(16 * ROWS_PER_S, ROW_TAIL)])
        plsc.subcore_barrier()


def _scatter(m, dsts):
    f = functools.partial(
        pl.kernel,
        out_type=jax.ShapeDtypeStruct((2, NET, N, H), _f32),
        mesh=plsc.VectorSubcoreMesh(core_axis_name="c", subcore_axis_name="s"),
        scratch_types=[
            pltpu.VMEM_SHARED((N, H), _f32),
            pltpu.VMEM((CH,), jnp.int32),
            pltpu.VMEM((CH,), jnp.int32),
            pltpu.VMEM((2, CH, H), _f32),
            pltpu.VMEM((8,), jnp.int32),
            pltpu.VMEM((8, H), _f32),
            pltpu.VMEM((ZR, H), _f32),
            pltpu.SemaphoreType.DMA,
            pltpu.SemaphoreType.DMA,
            pltpu.SemaphoreType.DMA,
            pltpu.SemaphoreType.DMA,
            pltpu.SemaphoreType.DMA,
            pltpu.SemaphoreType.DMA,
        ],
    )(_scatter_body)
    return f(m, *dsts)


def _gates1_body(x_ref, s_ref, wx_ref, wm_ref, h_ref, c_ref):
    x = x_ref[...].astype(jnp.bfloat16)
    g = jnp.dot(x, wx_ref[...], preferred_element_type=_f32)
    for et in range(NET):
        sm = (s_ref[0, et] + s_ref[1, et]).astype(jnp.bfloat16)
        g = g + jnp.dot(sm, wm_ref[et], preferred_element_type=_f32)
    i_g = jax.nn.sigmoid(g[:, 0:H])
    g_g = jnp.tanh(g[:, 2 * H:3 * H])
    o_g = jax.nn.sigmoid(g[:, 3 * H:4 * H])
    c_new = i_g * g_g
    c_ref[...] = c_new
    h_ref[...] = o_g * jnp.tanh(c_new)


def _gates1(x, s, wx, wm):
    nspec = pl.BlockSpec((NB, H), lambda t: (t, 0))
    return pl.pallas_call(
        _gates1_body,
        grid=(N // NB,),
        in_specs=[
            nspec,
            pl.BlockSpec((2, NET, NB, H), lambda t: (0, 0, t, 0)),
            pl.BlockSpec((H, 4 * H), lambda t: (0, 0)),
            pl.BlockSpec((NET, H, 4 * H), lambda t: (0, 0, 0)),
        ],
        out_specs=[nspec, nspec],
        out_shape=[jax.ShapeDtypeStruct((N, H), _f32)] * 2,
    )(x, s, wx.astype(jnp.bfloat16), wm.astype(jnp.bfloat16))


def _gates2_body(x_ref, s_ref, h_ref, c_ref, wx_ref, wm_ref, wh_ref, sw_ref,
                 lo_ref):
    x = x_ref[...].astype(jnp.bfloat16)
    g = jnp.dot(x, wx_ref[...], preferred_element_type=_f32)
    g = g + jnp.dot(h_ref[...].astype(jnp.bfloat16), wh_ref[...],
                    preferred_element_type=_f32)
    for et in range(NET):
        sm = (s_ref[0, et] + s_ref[1, et]).astype(jnp.bfloat16)
        g = g + jnp.dot(sm, wm_ref[et], preferred_element_type=_f32)
    i_g = jax.nn.sigmoid(g[:, 0:H])
    f_g = jax.nn.sigmoid(g[:, H:2 * H])
    g_g = jnp.tanh(g[:, 2 * H:3 * H])
    o_g = jax.nn.sigmoid(g[:, 3 * H:4 * H])
    c_new = f_g * c_ref[...] + i_g * g_g
    h_new = o_g * jnp.tanh(c_new)
    lo_ref[...] = jnp.dot(h_new.astype(jnp.bfloat16), sw_ref[...],
                          preferred_element_type=_f32)


def _gates2(x, s, h, cc, wx, wm, wh, swp):
    nspec = pl.BlockSpec((NB, H), lambda t: (t, 0))
    return pl.pallas_call(
        _gates2_body,
        grid=(N // NB,),
        in_specs=[
            nspec,
            pl.BlockSpec((2, NET, NB, H), lambda t: (0, 0, t, 0)),
            nspec,
            nspec,
            pl.BlockSpec((H, 4 * H), lambda t: (0, 0)),
            pl.BlockSpec((NET, H, 4 * H), lambda t: (0, 0, 0)),
            pl.BlockSpec((H, 4 * H), lambda t: (0, 0)),
            pl.BlockSpec((H, 8), lambda t: (0, 0)),
        ],
        out_specs=pl.BlockSpec((NB, 8), lambda t: (t, 0)),
        out_shape=jax.ShapeDtypeStruct((N, 8), _f32),
    )(x, s, h, cc, wx.astype(jnp.bfloat16), wm.astype(jnp.bfloat16),
      wh.astype(jnp.bfloat16), swp.astype(jnp.bfloat16))


def kernel(params, cell_idx, edge_index_intra_diff, edge_index_inter_diff,
           edge_index_intra_lt, edge_index_intra_gt):
    p = params
    ets = ('intra_diff', 'inter_diff', 'intra_lt', 'intra_gt')
    edges = (edge_index_intra_diff, edge_index_inter_diff,
             edge_index_intra_lt, edge_index_intra_gt)
    srcs = [e[0] for e in edges]
    dsts = [e[1] for e in edges]

    embp = jnp.zeros((8, H), _f32).at[:3, :].set(p['embed'])
    wpq = jnp.concatenate(
        [p['mlp_' + et]['W1'][:H] for et in ets]
        + [p['mlp_' + et]['W1'][H:] for et in ets], axis=1)
    w2s = jnp.stack([p['mlp_' + et]['W2'] for et in ets])
    w3s = jnp.stack([p['mlp_' + et]['W3'] for et in ets])
    w4s = jnp.stack([p['mlp_' + et]['W4'] for et in ets])
    b1s = jnp.stack([p['mlp_' + et]['b1'] for et in ets]).reshape(NET, 1, H)
    b2s = jnp.stack([p['mlp_' + et]['b2'] for et in ets]).reshape(NET, 1, H)
    b3s = jnp.stack([p['mlp_' + et]['b3'] for et in ets]).reshape(NET, 1, H)
    b4s = jnp.stack([p['mlp_' + et]['b4'] for et in ets]).reshape(NET, 1, H)
    wih = p['lstm_wih']
    wx = wih[:H]
    wm = wih[H:].reshape(NET, H, 4 * H)
    wh = p['lstm_whh']
    swp = jnp.zeros((H, 8), _f32).at[:, :1].set(p['score_w'])

    ci2 = jnp.broadcast_to(cell_idx.astype(jnp.int32)[:, None], (N, H))
    x = _embed(ci2, embp)

    h = x
    cc = None
    logits = None
    for step in range(STEPS):
        pqs = [t.reshape(N, 1, H // 2) for t in _pq(h, wpq)]
        zp, zq = _gather(pqs, srcs, dsts)
        m = _mlp(zp.reshape(NET, E, H // 2), zq.reshape(NET, E, H // 2),
                 w2s, w3s, w4s, b1s, b2s, b3s, b4s)
        s = _scatter(m, dsts)
        if step == 0:
            h, cc = _gates1(x, s, wx, wm)
        else:
            logits = _gates2(x, s, h, cc, wx, wm, wh, swp)
    return logits[:, 0]
